# baseline (speedup 1.0000x reference)
import dataclasses
import functools

import jax
import jax.numpy as jnp
from jax import lax
from jax.experimental import pallas as pl
from jax.experimental.pallas import tpu as pltpu
from jax.experimental.pallas import tpu_sc as plsc

L = 16
D = 768
KCH = D // L
TBLK = 32
G = 16
TT = 2048

N_SC = 10240


def _sc_body(n_sc, off_blocks, x_hbm, w_hbm, bv_hbm, o_hbm, w_v, bv_v, sem):
    pltpu.async_copy(w_hbm, w_v, sem).wait()
    pltpu.async_copy(bv_hbm, bv_v, sem).wait()
    bvec = bv_v[...]

    def block_body(x_vmem, o_vmem):
        @pl.loop(0, TBLK, step=G)
        def _(t0):
            def kstep(k, accs):
                wk = w_v[pl.ds(k * L, L)]
                return tuple(
                    accs[j] + x_vmem[t0 + j, pl.ds(k * L, L)] * wk
                    for j in range(G)
                )

            accs = lax.fori_loop(0, KCH, kstep, (bvec,) * G)
            lane = lax.iota(jnp.int32, L)
            r = jnp.zeros((L,), jnp.float32)
            for j in range(G):
                r = jnp.where(lane == j, jnp.sum(accs[j]), r)
            o_vmem[pl.ds(t0, G)] = r

    pltpu.emit_pipeline(
        block_body,
        grid=(n_sc // TBLK,),
        in_specs=[pl.BlockSpec((TBLK, D), lambda i: (i + off_blocks, 0))],
        out_specs=[pl.BlockSpec((TBLK,), lambda i: (i,))],
        core_axis_name=("c", "s"),
        dimension_semantics=(pltpu.PARALLEL,),
    )(x_hbm, o_hbm)


def _tc_body(x_ref, w_ref, b_ref, o_ref):
    o_ref[...] = jnp.sum(x_ref[...] * w_ref[...], axis=1) + b_ref[0]


def kernel(x, W, b):
    B, S, d = x.shape
    ntok = B * S
    x2 = x.reshape(ntok, d)

    n_sc = N_SC if ntok > N_SC else 0
    n_tc = ntok - n_sc

    out_tc = pl.pallas_call(
        _tc_body,
        grid=(n_tc // TT,),
        in_specs=[
            pl.BlockSpec((TT, d), lambda i: (i, 0)),
            pl.BlockSpec((1, d), lambda i: (0, 0)),
            pl.BlockSpec(memory_space=pltpu.SMEM),
        ],
        out_specs=pl.BlockSpec((TT,), lambda i: (i,)),
        out_shape=jax.ShapeDtypeStruct((n_tc,), jnp.float32),
    )(x2, W, b)

    if n_sc == 0:
        return out_tc.reshape(B, S)

    w = W.reshape(d)
    bv = jnp.concatenate([b.astype(jnp.float32), jnp.zeros((L - 1,), jnp.float32)])
    mesh = plsc.VectorSubcoreMesh(core_axis_name="c", subcore_axis_name="s")
    cp = pltpu.CompilerParams()
    if "needs_layout_passes" in pltpu.CompilerParams.__dataclass_fields__:
        cp = dataclasses.replace(cp, needs_layout_passes=False)
    out_sc = pl.kernel(
        functools.partial(_sc_body, n_sc, n_tc // TBLK),
        out_type=jax.ShapeDtypeStruct((n_sc,), jnp.float32),
        mesh=mesh,
        scratch_types=[
            pltpu.VMEM((D,), jnp.float32),
            pltpu.VMEM((L,), jnp.float32),
            pltpu.SemaphoreType.DMA,
        ],
        compiler_params=cp,
    )(x2, w, bv)

    return jnp.concatenate([out_tc, out_sc]).reshape(B, S)


# speedup vs baseline: 1.0039x; 1.0039x over previous
import dataclasses
import functools

import jax
import jax.numpy as jnp
from jax import lax
from jax.experimental import pallas as pl
from jax.experimental.pallas import tpu as pltpu
from jax.experimental.pallas import tpu_sc as plsc

L = 16
D = 768
KCH = D // L
TBLK = 32
G = 16
TT = 2048

N_SC = 10240


def _sc_body(n_sc, off_blocks, x_hbm, w_hbm, bv_hbm, o_hbm, w_v, bv_v, sem):
    pltpu.async_copy(w_hbm, w_v, sem).wait()
    pltpu.async_copy(bv_hbm, bv_v, sem).wait()
    bvec = bv_v[...]

    def block_body(x_vmem, o_vmem):
        @pl.loop(0, TBLK, step=G)
        def _(t0):
            def kstep(k, accs):
                wk = w_v[pl.ds(k * L, L)]
                return tuple(
                    accs[j] + x_vmem[t0 + j, pl.ds(k * L, L)] * wk
                    for j in range(G)
                )

            accs = lax.fori_loop(0, KCH, kstep, (bvec,) * G)
            lane = lax.iota(jnp.int32, L)
            r = jnp.zeros((L,), jnp.float32)
            for j in range(G):
                r = jnp.where(lane == j, jnp.sum(accs[j]), r)
            o_vmem[pl.ds(t0, G)] = r

    pltpu.emit_pipeline(
        block_body,
        grid=(n_sc // TBLK,),
        in_specs=[pl.BlockSpec((TBLK, D), lambda i: (i + off_blocks, 0))],
        out_specs=[pl.BlockSpec((TBLK,), lambda i: (i,))],
        core_axis_name=("c", "s"),
        dimension_semantics=(pltpu.PARALLEL,),
    )(x_hbm, o_hbm)


def _tc_body(x_ref, w_ref, b_ref, o_ref):
    o_ref[...] = jnp.sum(x_ref[...] * w_ref[...], axis=1) + b_ref[0]


def kernel(x, W, b):
    B, S, d = x.shape
    ntok = B * S
    x2 = x.reshape(ntok, d)

    n_sc = N_SC if ntok > N_SC else 0
    n_tc = ntok - n_sc

    out_sc = None
    if n_sc:
        w = W.reshape(d)
        bv = jnp.concatenate(
            [b.astype(jnp.float32), jnp.zeros((L - 1,), jnp.float32)]
        )
        mesh = plsc.VectorSubcoreMesh(core_axis_name="c", subcore_axis_name="s")
        cp = pltpu.CompilerParams()
        if "needs_layout_passes" in pltpu.CompilerParams.__dataclass_fields__:
            cp = dataclasses.replace(cp, needs_layout_passes=False)
        out_sc = pl.kernel(
            functools.partial(_sc_body, n_sc, n_tc // TBLK),
            out_type=jax.ShapeDtypeStruct((n_sc,), jnp.float32),
            mesh=mesh,
            scratch_types=[
                pltpu.VMEM((D,), jnp.float32),
                pltpu.VMEM((L,), jnp.float32),
                pltpu.SemaphoreType.DMA,
            ],
            compiler_params=cp,
        )(x2, w, bv)

    out_tc = pl.pallas_call(
        _tc_body,
        grid=(n_tc // TT,),
        in_specs=[
            pl.BlockSpec((TT, d), lambda i: (i, 0)),
            pl.BlockSpec((1, d), lambda i: (0, 0)),
            pl.BlockSpec(memory_space=pltpu.SMEM),
        ],
        out_specs=pl.BlockSpec((TT,), lambda i: (i,)),
        out_shape=jax.ShapeDtypeStruct((n_tc,), jnp.float32),
    )(x2, W, b)

    if out_sc is None:
        return out_tc.reshape(B, S)
    return jnp.concatenate([out_tc, out_sc]).reshape(B, S)
